# baseline (speedup 1.0000x reference)
import functools

import jax
import jax.numpy as jnp
from jax import lax
from jax.experimental import pallas as pl
from jax.experimental.pallas import tpu as pltpu
from jax.experimental.pallas import tpu_sc as plsc

D = 32
L = 50
B = 16384
V = 1000000
NC = 2
NS = 16
NW = NC * NS
BPW = B // NW
CB = 2
IPC = CB * L
NCHUNK = BPW // CB
NBUF = 4
INV_L = 1.0 / L


def _sc_body(ids_hbm, table_hbm, out_hbm, idx_v, rows_v, out_v, sems):
    c = lax.axis_index("c")
    s = lax.axis_index("s")
    w = s * NC + c

    pltpu.sync_copy(ids_hbm.at[w], idx_v)

    for b in range(NBUF):
        pltpu.async_copy(
            table_hbm.at[idx_v.at[b]], rows_v.at[b], sems.at[b]
        )

    def chunk_group(g, carry):
        for b in range(NBUF):
            j = g * NBUF + b
            pltpu.make_async_copy(
                table_hbm.at[idx_v.at[j]], rows_v.at[b], sems.at[b]
            ).wait()
            for e in range(CB):
                base = e * L
                a0 = rows_v[b, base, 0:16]
                a1 = rows_v[b, base, 16:32]
                for r in range(1, L):
                    a0 = a0 + rows_v[b, base + r, 0:16]
                    a1 = a1 + rows_v[b, base + r, 16:32]
                row = CB * j + e
                out_v[row, 0:16] = a0 * INV_L
                out_v[row, 16:32] = a1 * INV_L

            @pl.when(j + NBUF < NCHUNK)
            def _():
                pltpu.async_copy(
                    table_hbm.at[idx_v.at[j + NBUF]],
                    rows_v.at[b],
                    sems.at[b],
                )

        return carry

    lax.fori_loop(0, NCHUNK // NBUF, chunk_group, 0)

    pltpu.sync_copy(out_v, out_hbm.at[pl.ds(w * BPW, BPW)])


@functools.partial(
    pl.kernel,
    mesh=plsc.VectorSubcoreMesh(core_axis_name="c", subcore_axis_name="s"),
    out_type=jax.ShapeDtypeStruct((B, D), jnp.float32),
    scratch_types=[
        pltpu.VMEM((NCHUNK, IPC), jnp.int32),
        pltpu.VMEM((NBUF, IPC, 128), jnp.float32),
        pltpu.VMEM((BPW, D), jnp.float32),
        pltpu.SemaphoreType.DMA((NBUF,)),
    ],
    compiler_params=pltpu.CompilerParams(use_tc_tiling_on_sc=False),
)
def _mean_embed_sc(ids_hbm, table_hbm, out_hbm, idx_v, rows_v, out_v, sems):
    _sc_body(ids_hbm, table_hbm, out_hbm, idx_v, rows_v, out_v, sems)


def kernel(input_ids, table):
    tpad = jnp.pad(table, ((0, 0), (0, 128 - D)))
    ids = input_ids.astype(jnp.int32).reshape(NW, NCHUNK, IPC)
    return _mean_embed_sc(ids, tpad)


# speedup vs baseline: 1.1143x; 1.1143x over previous
import functools

import jax
import jax.numpy as jnp
from jax import lax
from jax.experimental import pallas as pl
from jax.experimental.pallas import tpu as pltpu
from jax.experimental.pallas import tpu_sc as plsc

D = 32
L = 50
B = 16384
V = 1000000
NC = 2
NS = 16
NW = NC * NS
BPW = B // NW
CB = 2
IPC = CB * L
NCHUNK = BPW // CB
NBUF = 8
INV_L = 1.0 / L


def _sc_body(ids_hbm, table_hbm, out_hbm, idx_v, rows_v, out_v, sems):
    c = lax.axis_index("c")
    s = lax.axis_index("s")
    w = s * NC + c

    pltpu.sync_copy(ids_hbm.at[w], idx_v)

    for b in range(NBUF):
        pltpu.async_copy(
            table_hbm.at[idx_v.at[b]], rows_v.at[b], sems.at[b]
        )

    def chunk_group(g, carry):
        for b in range(NBUF):
            j = g * NBUF + b
            pltpu.make_async_copy(
                table_hbm.at[idx_v.at[j]], rows_v.at[b], sems.at[b]
            ).wait()
            for e in range(CB):
                base = e * L
                a0 = rows_v[b, base, 0:16]
                a1 = rows_v[b, base, 16:32]
                for r in range(1, L):
                    a0 = a0 + rows_v[b, base + r, 0:16]
                    a1 = a1 + rows_v[b, base + r, 16:32]
                row = CB * j + e
                out_v[row, 0:16] = a0 * INV_L
                out_v[row, 16:32] = a1 * INV_L

            @pl.when(j + NBUF < NCHUNK)
            def _():
                pltpu.async_copy(
                    table_hbm.at[idx_v.at[j + NBUF]],
                    rows_v.at[b],
                    sems.at[b],
                )

        return carry

    lax.fori_loop(0, NCHUNK // NBUF, chunk_group, 0)

    pltpu.sync_copy(out_v, out_hbm.at[pl.ds(w * BPW, BPW)])


@functools.partial(
    pl.kernel,
    mesh=plsc.VectorSubcoreMesh(core_axis_name="c", subcore_axis_name="s"),
    out_type=jax.ShapeDtypeStruct((B, D), jnp.float32),
    scratch_types=[
        pltpu.VMEM((NCHUNK, IPC), jnp.int32),
        pltpu.VMEM((NBUF, IPC, D), jnp.float32),
        pltpu.VMEM((BPW, D), jnp.float32),
        pltpu.SemaphoreType.DMA((NBUF,)),
    ],
    compiler_params=pltpu.CompilerParams(use_tc_tiling_on_sc=False),
)
def _mean_embed_sc(ids_hbm, table_hbm, out_hbm, idx_v, rows_v, out_v, sems):
    _sc_body(ids_hbm, table_hbm, out_hbm, idx_v, rows_v, out_v, sems)


def kernel(input_ids, table):
    t128 = jnp.reshape(table, (V * D // 128, 128))
    t128 = lax.optimization_barrier(t128)
    table_lin = jnp.reshape(t128, (V, D))
    ids = input_ids.astype(jnp.int32).reshape(NW, NCHUNK, IPC)
    return _mean_embed_sc(ids, table_lin)


# speedup vs baseline: 1.1540x; 1.0356x over previous
import functools

import jax
import jax.numpy as jnp
from jax import lax
from jax.experimental import pallas as pl
from jax.experimental.pallas import tpu as pltpu
from jax.experimental.pallas import tpu_sc as plsc

D = 32
L = 50
B = 16384
V = 1000000
NC = 2
NS = 16
NW = NC * NS
BPW = B // NW
CB = 2
IPC = CB * L
NCHUNK = BPW // CB
NBUF = 4
INV_L = 1.0 / L


def _sc_body(ids_hbm, table_hbm, out_hbm, idx_v, rows_v, out_v, sems):
    c = lax.axis_index("c")
    s = lax.axis_index("s")
    w = s * NC + c

    pltpu.sync_copy(ids_hbm.at[w], idx_v)

    for b in range(NBUF):
        pltpu.async_copy(
            table_hbm.at[idx_v.at[b]], rows_v.at[b], sems.at[b]
        )

    def chunk_group(g, carry):
        for b in range(NBUF):
            j = g * NBUF + b
            pltpu.make_async_copy(
                table_hbm.at[idx_v.at[j]], rows_v.at[b], sems.at[b]
            ).wait()
            for e in range(CB):
                base = e * L
                a0 = rows_v[b, base, 0:16]
                a1 = rows_v[b, base, 16:32]
                for r in range(1, L):
                    a0 = a0 + rows_v[b, base + r, 0:16]
                    a1 = a1 + rows_v[b, base + r, 16:32]
                row = CB * j + e
                out_v[row, 0:16] = a0 * INV_L
                out_v[row, 16:32] = a1 * INV_L

            @pl.when(j + NBUF < NCHUNK)
            def _():
                pltpu.async_copy(
                    table_hbm.at[idx_v.at[j + NBUF]],
                    rows_v.at[b],
                    sems.at[b],
                )

        return carry

    lax.fori_loop(0, NCHUNK // NBUF, chunk_group, 0)

    pltpu.sync_copy(out_v, out_hbm.at[pl.ds(w * BPW, BPW)])


@functools.partial(
    pl.kernel,
    mesh=plsc.VectorSubcoreMesh(core_axis_name="c", subcore_axis_name="s"),
    out_type=jax.ShapeDtypeStruct((B, D), jnp.float32),
    scratch_types=[
        pltpu.VMEM((NCHUNK, IPC), jnp.int32),
        pltpu.VMEM((NBUF, IPC, D), jnp.float32),
        pltpu.VMEM((BPW, D), jnp.float32),
        pltpu.SemaphoreType.DMA((NBUF,)),
    ],
    compiler_params=pltpu.CompilerParams(use_tc_tiling_on_sc=False),
)
def _mean_embed_sc(ids_hbm, table_hbm, out_hbm, idx_v, rows_v, out_v, sems):
    _sc_body(ids_hbm, table_hbm, out_hbm, idx_v, rows_v, out_v, sems)


def kernel(input_ids, table):
    t128 = jnp.reshape(table, (V * D // 128, 128))
    t128 = lax.optimization_barrier(t128)
    table_lin = jnp.reshape(t128, (V, D))
    ids = input_ids.astype(jnp.int32).reshape(NW, NCHUNK, IPC)
    return _mean_embed_sc(ids, table_lin)
